# baseline (speedup 1.0000x reference)
import functools

import jax
import jax.numpy as jnp
from jax import lax
from jax.experimental import pallas as pl
from jax.experimental.pallas import tpu as pltpu
from jax.experimental.pallas import tpu_sc as plsc

T, D, F, E, K = 2048, 1024, 2048, 8, 2
TM = 128
P = K * T + E * TM
NT = P // TM
RCH = 512
NRCH = (K * T) // RCH

NC, NS = 2, 16
NW = NC * NS
APW = (K * T) // NW
DCH = 64
TPW = T // NW
CCH = 32


def _router_body(x_ref, wr_ref, pos_ref, gcat_ref, meta_ref, laux_ref,
                 xbf_ref):
    x = x_ref[...]
    xbf_ref[...] = x.astype(jnp.bfloat16)
    wr = wr_ref[...]
    logits = lax.dot_general(x, wr, (((1,), (1,)), ((), ())),
                             preferred_element_type=jnp.float32)
    m = jnp.max(logits, axis=-1, keepdims=True)
    ex = jnp.exp(logits - m)
    probs = ex / jnp.sum(ex, axis=-1, keepdims=True)

    iota = lax.broadcasted_iota(jnp.int32, (T, E), 1)
    m1 = jnp.max(probs, axis=-1, keepdims=True)
    i1 = jnp.min(jnp.where(probs == m1, iota, E), axis=-1, keepdims=True)
    masked = jnp.where(iota == i1, -1.0, probs)
    m2 = jnp.max(masked, axis=-1, keepdims=True)
    i2 = jnp.min(jnp.where(masked == m2, iota, E), axis=-1, keepdims=True)
    denom = m1 + m2
    gcat_ref[0:T, :] = jnp.broadcast_to(m1 / denom, (T, 128))
    gcat_ref[T:K * T, :] = jnp.broadcast_to(m2 / denom, (T, 128))

    sel1 = (iota == i1).astype(jnp.float32)
    sel2 = (iota == i2).astype(jnp.float32)

    f = jnp.sum(sel1 + sel2, axis=0, keepdims=True) / T
    p = jnp.sum(probs, axis=0, keepdims=True) / T
    laux_ref[...] = jnp.sum(E * f * p, axis=-1, keepdims=True)

    counts = jnp.sum(sel1, axis=0, keepdims=True) + jnp.sum(
        sel2, axis=0, keepdims=True)
    pc = jnp.floor((counts + (TM - 1)) / TM) * TM
    er = lax.broadcasted_iota(jnp.int32, (E, E), 0)
    ec = lax.broadcasted_iota(jnp.int32, (E, E), 1)
    upper = (er < ec).astype(jnp.float32)
    start = lax.dot_general(pc, upper, (((1,), (0,)), ((), ())),
                            preferred_element_type=jnp.float32)
    pend = start + pc
    pend_total = jnp.sum(pc, axis=-1, keepdims=True)

    onehot = jnp.concatenate([sel1, sel2], axis=0)
    rr = lax.broadcasted_iota(jnp.int32, (RCH, RCH), 0)
    rc = lax.broadcasted_iota(jnp.int32, (RCH, RCH), 1)
    tril = (rc < rr).astype(jnp.float32)
    base = jnp.zeros((1, E), jnp.float32)
    for c in range(NRCH):
        oc = onehot[c * RCH:(c + 1) * RCH, :]
        run = lax.dot_general(tril, oc, (((1,), (0,)), ((), ())),
                              preferred_element_type=jnp.float32) + base
        base = base + jnp.sum(oc, axis=0, keepdims=True)
        rank = jnp.sum(run * oc, axis=-1, keepdims=True)
        st = jnp.sum(start * oc, axis=-1, keepdims=True)
        pos_ref[c * RCH:(c + 1) * RCH, :] = (rank + st).astype(jnp.int32)

    ie8 = lax.broadcasted_iota(jnp.int32, (1, E), 1).astype(jnp.float32)
    present = (pc > 0).astype(jnp.float32)
    lastp = jnp.max(jnp.where(pc > 0, ie8, -1.0), axis=-1,
                    keepdims=True)

    ti = (lax.broadcasted_iota(jnp.int32, (1, 128), 1) * TM).astype(
        jnp.float32)
    te = jnp.zeros((1, 128), jnp.float32)
    for e in range(E):
        te = te + (ti >= pend[:, e:e + 1]).astype(jnp.float32)
    te = jnp.minimum(te, float(E - 1))
    te = jnp.where(ti < pend_total, te, lastp)

    startmap = jnp.zeros((1, 128), jnp.float32)
    rankmap = jnp.zeros((1, 128), jnp.float32)
    nexte = jnp.full((1, 128), float(E), jnp.float32)
    for e in range(E):
        sel = (te == float(e)).astype(jnp.float32)
        startmap = startmap + sel * start[:, e:e + 1]
        rankmap = rankmap + jnp.where(
            (present[:, e:e + 1] > 0) & (te >= float(e)), 1.0, 0.0)
        cand = jnp.where((present[:, e:e + 1] > 0) & (te < float(e)),
                         float(e), float(E))
        nexte = jnp.minimum(nexte, cand)
    nexte = jnp.where(nexte == float(E), te, nexte)
    slot = rankmap - 1.0
    slot = slot - 2.0 * jnp.floor(slot * 0.5)
    runstart = (ti == startmap).astype(jnp.float32)
    prestart = runstart * (te != lastp).astype(jnp.float32)

    meta_ref[0:1, :] = te.astype(jnp.int32)
    meta_ref[1:2, :] = slot.astype(jnp.int32)
    meta_ref[2:3, :] = nexte.astype(jnp.int32)
    meta_ref[3:4, :] = runstart.astype(jnp.int32)
    meta_ref[4:5, :] = prestart.astype(jnp.int32)


def _dispatch_body(x_hbm, pos_hbm, gcat_hbm, xg_hbm, gq_hbm,
                   idx_v, xbuf, gbuf, sem):
    wid = lax.axis_index("s") * NC + lax.axis_index("c")
    for sub in range(APW // DCH):
        j0 = wid * APW + sub * DCH
        t0 = lax.rem(j0, T)
        pltpu.sync_copy(pos_hbm.at[pl.ds(j0, DCH)], idx_v)
        pltpu.sync_copy(x_hbm.at[pl.ds(t0, DCH)], xbuf)
        pltpu.sync_copy(gcat_hbm.at[pl.ds(j0, DCH)], gbuf)
        pltpu.async_copy(xbuf, xg_hbm.at[idx_v], sem).wait()
        pltpu.async_copy(gbuf, gq_hbm.at[idx_v], sem).wait()


def _combine_body(og_hbm, p1_hbm, p2_hbm, y_hbm, i1v, i2v, b1, b2, s1, s2):
    wid = lax.axis_index("s") * NC + lax.axis_index("c")
    for sub in range(TPW // CCH):
        t0 = wid * TPW + sub * CCH
        pltpu.sync_copy(p1_hbm.at[pl.ds(t0, CCH)], i1v)
        pltpu.sync_copy(p2_hbm.at[pl.ds(t0, CCH)], i2v)
        c1 = pltpu.async_copy(og_hbm.at[i1v], b1, s1)
        c2 = pltpu.async_copy(og_hbm.at[i2v], b2, s2)
        c1.wait()
        c2.wait()

        def row_body(r, carry):
            def col_body(cc, carry2):
                off = cc * 64
                for u in range(4):
                    sl = pl.ds(off + u * 16, 16)
                    b1[r, sl] = b1[r, sl] + b2[r, sl]
                return carry2

            return lax.fori_loop(0, D // 64, col_body, carry)

        lax.fori_loop(0, CCH, row_body, 0)
        pltpu.sync_copy(b1, y_hbm.at[pl.ds(t0, CCH)])


@functools.lru_cache(maxsize=None)
def _sc_kernels():
    mesh = plsc.VectorSubcoreMesh(core_axis_name="c", subcore_axis_name="s")
    dispatch = pl.kernel(
        _dispatch_body,
        out_type=(
            jax.ShapeDtypeStruct((P, D // 2), jnp.int32),
            jax.ShapeDtypeStruct((P, 128), jnp.float32),
        ),
        mesh=mesh,
        scratch_types=[
            pltpu.VMEM((DCH,), jnp.int32),
            pltpu.VMEM((DCH, D // 2), jnp.int32),
            pltpu.VMEM((DCH, 128), jnp.float32),
            pltpu.SemaphoreType.DMA,
        ],
    )
    combine = pl.kernel(
        _combine_body,
        out_type=jax.ShapeDtypeStruct((T, D), jnp.float32),
        mesh=mesh,
        scratch_types=[
            pltpu.VMEM((CCH,), jnp.int32),
            pltpu.VMEM((CCH,), jnp.int32),
            pltpu.VMEM((CCH, D), jnp.float32),
            pltpu.VMEM((CCH, D), jnp.float32),
            pltpu.SemaphoreType.DMA,
            pltpu.SemaphoreType.DMA,
        ],
    )
    return dispatch, combine


def _gmm_body(meta_ref, xg_ref, w1_hbm, w2_hbm, b1_ref, b2_ref, gq_ref,
              og_ref, w1a, w1b, w2a, w2b, sw1a, sw1b, sw2a, sw2b):
    i = pl.program_id(0)
    e = meta_ref[0, i]
    slot = meta_ref[1, i]
    nxt = meta_ref[2, i]
    rs = meta_ref[3, i]
    ps = meta_ref[4, i]

    @pl.when(i == 0)
    def _():
        pltpu.make_async_copy(w1_hbm.at[e], w1a, sw1a).start()
        pltpu.make_async_copy(w2_hbm.at[e], w2a, sw2a).start()

    @pl.when((ps == 1) & (slot == 0))
    def _():
        pltpu.make_async_copy(w1_hbm.at[nxt], w1b, sw1b).start()
        pltpu.make_async_copy(w2_hbm.at[nxt], w2b, sw2b).start()

    @pl.when((ps == 1) & (slot == 1))
    def _():
        pltpu.make_async_copy(w1_hbm.at[nxt], w1a, sw1a).start()
        pltpu.make_async_copy(w2_hbm.at[nxt], w2a, sw2a).start()

    @pl.when((rs == 1) & (slot == 0))
    def _():
        pltpu.make_async_copy(w1_hbm.at[e], w1a, sw1a).wait()
        pltpu.make_async_copy(w2_hbm.at[e], w2a, sw2a).wait()

    @pl.when((rs == 1) & (slot == 1))
    def _():
        pltpu.make_async_copy(w1_hbm.at[e], w1b, sw1b).wait()
        pltpu.make_async_copy(w2_hbm.at[e], w2b, sw2b).wait()

    def compute(w1buf, w2buf):
        xb = xg_ref[...]
        h = lax.dot_general(xb, w1buf[...].astype(jnp.bfloat16),
                            (((1,), (1,)), ((), ())),
                            preferred_element_type=jnp.float32)
        h = jnp.maximum(h + b1_ref[0], 0.0).astype(jnp.bfloat16)
        o = lax.dot_general(h, w2buf[...].astype(jnp.bfloat16),
                            (((1,), (1,)), ((), ())),
                            preferred_element_type=jnp.float32)
        og_ref[...] = (o + b2_ref[0]) * gq_ref[:, 0:1]

    @pl.when(slot == 0)
    def _():
        compute(w1a, w2a)

    @pl.when(slot == 1)
    def _():
        compute(w1b, w2b)


@jax.jit
def _moe(x, Wr, W1, b1, W2, b2):
    pos, gcat, meta, laux, xbf = pl.pallas_call(
        _router_body,
        out_shape=(
            jax.ShapeDtypeStruct((K * T, 1), jnp.int32),
            jax.ShapeDtypeStruct((K * T, 128), jnp.float32),
            jax.ShapeDtypeStruct((5, 128), jnp.int32),
            jax.ShapeDtypeStruct((1, 1), jnp.float32),
            jax.ShapeDtypeStruct((T, D), jnp.bfloat16),
        ),
    )(x, Wr)

    dispatch, combine = _sc_kernels()
    pos_flat = pos.reshape(K * T)
    xbf_i32 = lax.bitcast_convert_type(
        xbf.reshape(T, D // 2, 2), jnp.int32)
    xg_i32, gq = dispatch(xbf_i32, pos_flat, gcat)
    xg = lax.bitcast_convert_type(xg_i32, jnp.bfloat16).reshape(P, D)

    grid_spec = pltpu.PrefetchScalarGridSpec(
        num_scalar_prefetch=1,
        grid=(NT,),
        in_specs=[
            pl.BlockSpec((TM, D), lambda i, m: (i, 0)),
            pl.BlockSpec(memory_space=pl.MemorySpace.ANY),
            pl.BlockSpec(memory_space=pl.MemorySpace.ANY),
            pl.BlockSpec((1, 1, F), lambda i, m: (m[0, i], 0, 0)),
            pl.BlockSpec((1, 1, D), lambda i, m: (m[0, i], 0, 0)),
            pl.BlockSpec((TM, 128), lambda i, m: (i, 0)),
        ],
        out_specs=pl.BlockSpec((TM, D), lambda i, m: (i, 0)),
        scratch_shapes=[
            pltpu.VMEM((F, D), jnp.float32),
            pltpu.VMEM((F, D), jnp.float32),
            pltpu.VMEM((D, F), jnp.float32),
            pltpu.VMEM((D, F), jnp.float32),
            pltpu.SemaphoreType.DMA,
            pltpu.SemaphoreType.DMA,
            pltpu.SemaphoreType.DMA,
            pltpu.SemaphoreType.DMA,
        ],
    )
    og = pl.pallas_call(
        _gmm_body,
        grid_spec=grid_spec,
        out_shape=jax.ShapeDtypeStruct((P, D), jnp.float32),
    )(meta, xg, W1, W2, b1.reshape(E, 1, F), b2.reshape(E, 1, D), gq)

    y = combine(og, pos_flat[:T], pos_flat[T:])
    return y, laux[0, 0]


def kernel(x, Wr, W1, b1, W2, b2):
    return _moe(x, Wr, W1, b1, W2, b2)


# speedup vs baseline: 1.7612x; 1.7612x over previous
import functools

import jax
import jax.numpy as jnp
from jax import lax
from jax.experimental import pallas as pl
from jax.experimental.pallas import tpu as pltpu
from jax.experimental.pallas import tpu_sc as plsc

T, D, F, E, K = 2048, 1024, 2048, 8, 2
TM = 128
P = K * T + E * TM
NT = P // TM
RCH = 512
NRCH = (K * T) // RCH

NC, NS = 2, 16
NW = NC * NS
APW = (K * T) // NW
DCH = 64
TPW = T // NW
CCH = 32


def _router_body(x_ref, wr_ref, pos_ref, gcat_ref, meta_ref, laux_ref):
    x = x_ref[...]
    wr = wr_ref[...]
    logits = lax.dot_general(x, wr, (((1,), (1,)), ((), ())),
                             preferred_element_type=jnp.float32)
    m = jnp.max(logits, axis=-1, keepdims=True)
    ex = jnp.exp(logits - m)
    probs = ex / jnp.sum(ex, axis=-1, keepdims=True)

    iota = lax.broadcasted_iota(jnp.int32, (T, E), 1)
    m1 = jnp.max(probs, axis=-1, keepdims=True)
    i1 = jnp.min(jnp.where(probs == m1, iota, E), axis=-1, keepdims=True)
    masked = jnp.where(iota == i1, -1.0, probs)
    m2 = jnp.max(masked, axis=-1, keepdims=True)
    i2 = jnp.min(jnp.where(masked == m2, iota, E), axis=-1, keepdims=True)
    denom = m1 + m2
    gcat_ref[0:T, :] = jnp.broadcast_to(m1 / denom, (T, 128))
    gcat_ref[T:K * T, :] = jnp.broadcast_to(m2 / denom, (T, 128))

    sel1 = (iota == i1).astype(jnp.float32)
    sel2 = (iota == i2).astype(jnp.float32)

    f = jnp.sum(sel1 + sel2, axis=0, keepdims=True) / T
    p = jnp.sum(probs, axis=0, keepdims=True) / T
    laux_ref[...] = jnp.sum(E * f * p, axis=-1, keepdims=True)

    counts = jnp.sum(sel1, axis=0, keepdims=True) + jnp.sum(
        sel2, axis=0, keepdims=True)
    pc = jnp.floor((counts + (TM - 1)) / TM) * TM
    er = lax.broadcasted_iota(jnp.int32, (E, E), 0)
    ec = lax.broadcasted_iota(jnp.int32, (E, E), 1)
    upper = (er < ec).astype(jnp.float32)
    start = lax.dot_general(pc, upper, (((1,), (0,)), ((), ())),
                            preferred_element_type=jnp.float32)
    pend = start + pc
    pend_total = jnp.sum(pc, axis=-1, keepdims=True)

    onehot = jnp.concatenate([sel1, sel2], axis=0)
    rr = lax.broadcasted_iota(jnp.int32, (RCH, RCH), 0)
    rc = lax.broadcasted_iota(jnp.int32, (RCH, RCH), 1)
    tril = (rc < rr).astype(jnp.float32)
    base = jnp.zeros((1, E), jnp.float32)
    for c in range(NRCH):
        oc = onehot[c * RCH:(c + 1) * RCH, :]
        run = lax.dot_general(tril, oc, (((1,), (0,)), ((), ())),
                              preferred_element_type=jnp.float32) + base
        base = base + jnp.sum(oc, axis=0, keepdims=True)
        rank = jnp.sum(run * oc, axis=-1, keepdims=True)
        st = jnp.sum(start * oc, axis=-1, keepdims=True)
        pos_ref[c * RCH:(c + 1) * RCH, :] = (rank + st).astype(jnp.int32)

    ie8 = lax.broadcasted_iota(jnp.int32, (1, E), 1).astype(jnp.float32)
    present = (pc > 0).astype(jnp.float32)
    lastp = jnp.max(jnp.where(pc > 0, ie8, -1.0), axis=-1,
                    keepdims=True)

    ti = (lax.broadcasted_iota(jnp.int32, (1, 128), 1) * TM).astype(
        jnp.float32)
    te = jnp.zeros((1, 128), jnp.float32)
    for e in range(E):
        te = te + (ti >= pend[:, e:e + 1]).astype(jnp.float32)
    te = jnp.minimum(te, float(E - 1))
    te = jnp.where(ti < pend_total, te, lastp)

    startmap = jnp.zeros((1, 128), jnp.float32)
    rankmap = jnp.zeros((1, 128), jnp.float32)
    nexte = jnp.full((1, 128), float(E), jnp.float32)
    for e in range(E):
        sel = (te == float(e)).astype(jnp.float32)
        startmap = startmap + sel * start[:, e:e + 1]
        rankmap = rankmap + jnp.where(
            (present[:, e:e + 1] > 0) & (te >= float(e)), 1.0, 0.0)
        cand = jnp.where((present[:, e:e + 1] > 0) & (te < float(e)),
                         float(e), float(E))
        nexte = jnp.minimum(nexte, cand)
    nexte = jnp.where(nexte == float(E), te, nexte)
    slot = rankmap - 1.0
    slot = slot - 2.0 * jnp.floor(slot * 0.5)
    runstart = (ti == startmap).astype(jnp.float32)
    prestart = runstart * (te != lastp).astype(jnp.float32)

    meta_ref[0:1, :] = te.astype(jnp.int32)
    meta_ref[1:2, :] = slot.astype(jnp.int32)
    meta_ref[2:3, :] = nexte.astype(jnp.int32)
    meta_ref[3:4, :] = runstart.astype(jnp.int32)
    meta_ref[4:5, :] = prestart.astype(jnp.int32)


def _dispatch_body(x_hbm, pos_hbm, gcat_hbm, xg_hbm, gq_hbm,
                   idx_v, xbuf, gbuf, sem):
    wid = lax.axis_index("s") * NC + lax.axis_index("c")
    for sub in range(APW // DCH):
        j0 = wid * APW + sub * DCH
        t0 = lax.rem(j0, T)
        pltpu.sync_copy(pos_hbm.at[pl.ds(j0, DCH)], idx_v)
        pltpu.sync_copy(x_hbm.at[pl.ds(t0, DCH)], xbuf)
        pltpu.sync_copy(gcat_hbm.at[pl.ds(j0, DCH)], gbuf)
        pltpu.async_copy(xbuf, xg_hbm.at[idx_v], sem).wait()
        pltpu.async_copy(gbuf, gq_hbm.at[idx_v], sem).wait()


def _combine_body(og_hbm, p1_hbm, p2_hbm, y_hbm, i1v, i2v, b1, b2, s1, s2):
    wid = lax.axis_index("s") * NC + lax.axis_index("c")
    for sub in range(TPW // CCH):
        t0 = wid * TPW + sub * CCH
        pltpu.sync_copy(p1_hbm.at[pl.ds(t0, CCH)], i1v)
        pltpu.sync_copy(p2_hbm.at[pl.ds(t0, CCH)], i2v)
        c1 = pltpu.async_copy(og_hbm.at[i1v], b1, s1)
        c2 = pltpu.async_copy(og_hbm.at[i2v], b2, s2)
        c1.wait()
        c2.wait()

        def row_body(r, carry):
            def col_body(cc, carry2):
                off = cc * 64
                for u in range(4):
                    sl = pl.ds(off + u * 16, 16)
                    b1[r, sl] = b1[r, sl] + b2[r, sl]
                return carry2

            return lax.fori_loop(0, D // 64, col_body, carry)

        lax.fori_loop(0, CCH, row_body, 0)
        pltpu.sync_copy(b1, y_hbm.at[pl.ds(t0, CCH)])


@functools.lru_cache(maxsize=None)
def _sc_kernels():
    mesh = plsc.VectorSubcoreMesh(core_axis_name="c", subcore_axis_name="s")
    dispatch = pl.kernel(
        _dispatch_body,
        out_type=(
            jax.ShapeDtypeStruct((P, D), jnp.float32),
            jax.ShapeDtypeStruct((P, 128), jnp.float32),
        ),
        mesh=mesh,
        scratch_types=[
            pltpu.VMEM((DCH,), jnp.int32),
            pltpu.VMEM((DCH, D), jnp.float32),
            pltpu.VMEM((DCH, 128), jnp.float32),
            pltpu.SemaphoreType.DMA,
        ],
    )
    combine = pl.kernel(
        _combine_body,
        out_type=jax.ShapeDtypeStruct((T, D), jnp.float32),
        mesh=mesh,
        scratch_types=[
            pltpu.VMEM((CCH,), jnp.int32),
            pltpu.VMEM((CCH,), jnp.int32),
            pltpu.VMEM((CCH, D), jnp.float32),
            pltpu.VMEM((CCH, D), jnp.float32),
            pltpu.SemaphoreType.DMA,
            pltpu.SemaphoreType.DMA,
        ],
    )
    return dispatch, combine


def _gmm_body(meta_ref, xg_ref, w1_hbm, w2_hbm, b1_ref, b2_ref, gq_ref,
              og_ref, w1a, w1b, w2a, w2b, sw1a, sw1b, sw2a, sw2b):
    i = pl.program_id(0)
    e = meta_ref[0, i]
    slot = meta_ref[1, i]
    nxt = meta_ref[2, i]
    rs = meta_ref[3, i]
    ps = meta_ref[4, i]

    @pl.when(i == 0)
    def _():
        pltpu.make_async_copy(w1_hbm.at[e], w1a, sw1a).start()
        pltpu.make_async_copy(w2_hbm.at[e], w2a, sw2a).start()

    @pl.when((ps == 1) & (slot == 0))
    def _():
        pltpu.make_async_copy(w1_hbm.at[nxt], w1b, sw1b).start()
        pltpu.make_async_copy(w2_hbm.at[nxt], w2b, sw2b).start()

    @pl.when((ps == 1) & (slot == 1))
    def _():
        pltpu.make_async_copy(w1_hbm.at[nxt], w1a, sw1a).start()
        pltpu.make_async_copy(w2_hbm.at[nxt], w2a, sw2a).start()

    @pl.when((rs == 1) & (slot == 0))
    def _():
        pltpu.make_async_copy(w1_hbm.at[e], w1a, sw1a).wait()
        pltpu.make_async_copy(w2_hbm.at[e], w2a, sw2a).wait()

    @pl.when((rs == 1) & (slot == 1))
    def _():
        pltpu.make_async_copy(w1_hbm.at[e], w1b, sw1b).wait()
        pltpu.make_async_copy(w2_hbm.at[e], w2b, sw2b).wait()

    def compute(w1buf, w2buf):
        xb = xg_ref[...].astype(jnp.bfloat16)
        h = lax.dot_general(xb, w1buf[...].astype(jnp.bfloat16),
                            (((1,), (1,)), ((), ())),
                            preferred_element_type=jnp.float32)
        h = jnp.maximum(h + b1_ref[0], 0.0).astype(jnp.bfloat16)
        o = lax.dot_general(h, w2buf[...].astype(jnp.bfloat16),
                            (((1,), (1,)), ((), ())),
                            preferred_element_type=jnp.float32)
        og_ref[...] = (o + b2_ref[0]) * gq_ref[:, 0:1]

    @pl.when(slot == 0)
    def _():
        compute(w1a, w2a)

    @pl.when(slot == 1)
    def _():
        compute(w1b, w2b)


@jax.jit
def _moe(x, Wr, W1, b1, W2, b2):
    pos, gcat, meta, laux = pl.pallas_call(
        _router_body,
        out_shape=(
            jax.ShapeDtypeStruct((K * T, 1), jnp.int32),
            jax.ShapeDtypeStruct((K * T, 128), jnp.float32),
            jax.ShapeDtypeStruct((5, 128), jnp.int32),
            jax.ShapeDtypeStruct((1, 1), jnp.float32),
        ),
    )(x, Wr)

    dispatch, combine = _sc_kernels()
    pos_flat = pos.reshape(K * T)
    xg, gq = dispatch(x, pos_flat, gcat)

    grid_spec = pltpu.PrefetchScalarGridSpec(
        num_scalar_prefetch=1,
        grid=(NT,),
        in_specs=[
            pl.BlockSpec((TM, D), lambda i, m: (i, 0)),
            pl.BlockSpec(memory_space=pl.MemorySpace.ANY),
            pl.BlockSpec(memory_space=pl.MemorySpace.ANY),
            pl.BlockSpec((1, 1, F), lambda i, m: (m[0, i], 0, 0)),
            pl.BlockSpec((1, 1, D), lambda i, m: (m[0, i], 0, 0)),
            pl.BlockSpec((TM, 128), lambda i, m: (i, 0)),
        ],
        out_specs=pl.BlockSpec((TM, D), lambda i, m: (i, 0)),
        scratch_shapes=[
            pltpu.VMEM((F, D), jnp.float32),
            pltpu.VMEM((F, D), jnp.float32),
            pltpu.VMEM((D, F), jnp.float32),
            pltpu.VMEM((D, F), jnp.float32),
            pltpu.SemaphoreType.DMA,
            pltpu.SemaphoreType.DMA,
            pltpu.SemaphoreType.DMA,
            pltpu.SemaphoreType.DMA,
        ],
    )
    og = pl.pallas_call(
        _gmm_body,
        grid_spec=grid_spec,
        out_shape=jax.ShapeDtypeStruct((P, D), jnp.float32),
    )(meta, xg, W1, W2, b1.reshape(E, 1, F), b2.reshape(E, 1, D), gq)

    y = combine(og, pos_flat[:T], pos_flat[T:])
    return y, laux[0, 0]


def kernel(x, Wr, W1, b1, W2, b2):
    return _moe(x, Wr, W1, b1, W2, b2)


# speedup vs baseline: 2.4087x; 1.3677x over previous
import functools

import jax
import jax.numpy as jnp
from jax import lax
from jax.experimental import pallas as pl
from jax.experimental.pallas import tpu as pltpu
from jax.experimental.pallas import tpu_sc as plsc

T, D, F, E, K = 2048, 1024, 2048, 8, 2
TM = 256
P = K * T + E * TM
NT = P // TM
RCH = 512
NRCH = (K * T) // RCH

NC, NS = 2, 16
NW = NC * NS
APW = (K * T) // NW
DCH = 64
TPW = T // NW
CCH = 32


def _router_body(x_ref, wr_ref, pos_ref, gcat_ref, meta_ref, laux_ref):
    x = x_ref[...]
    wr = wr_ref[...]
    logits = lax.dot_general(x, wr, (((1,), (1,)), ((), ())),
                             preferred_element_type=jnp.float32)
    m = jnp.max(logits, axis=-1, keepdims=True)
    ex = jnp.exp(logits - m)
    probs = ex / jnp.sum(ex, axis=-1, keepdims=True)

    iota = lax.broadcasted_iota(jnp.int32, (T, E), 1)
    m1 = jnp.max(probs, axis=-1, keepdims=True)
    i1 = jnp.min(jnp.where(probs == m1, iota, E), axis=-1, keepdims=True)
    masked = jnp.where(iota == i1, -1.0, probs)
    m2 = jnp.max(masked, axis=-1, keepdims=True)
    i2 = jnp.min(jnp.where(masked == m2, iota, E), axis=-1, keepdims=True)
    denom = m1 + m2
    gcat_ref[0:T, :] = jnp.broadcast_to(m1 / denom, (T, 128))
    gcat_ref[T:K * T, :] = jnp.broadcast_to(m2 / denom, (T, 128))

    sel1 = (iota == i1).astype(jnp.float32)
    sel2 = (iota == i2).astype(jnp.float32)

    f = jnp.sum(sel1 + sel2, axis=0, keepdims=True) / T
    p = jnp.sum(probs, axis=0, keepdims=True) / T
    laux_ref[...] = jnp.sum(E * f * p, axis=-1, keepdims=True)

    counts = jnp.sum(sel1, axis=0, keepdims=True) + jnp.sum(
        sel2, axis=0, keepdims=True)
    pc = jnp.floor((counts + (TM - 1)) / TM) * TM
    er = lax.broadcasted_iota(jnp.int32, (E, E), 0)
    ec = lax.broadcasted_iota(jnp.int32, (E, E), 1)
    upper = (er < ec).astype(jnp.float32)
    start = lax.dot_general(pc, upper, (((1,), (0,)), ((), ())),
                            preferred_element_type=jnp.float32)
    pend = start + pc
    pend_total = jnp.sum(pc, axis=-1, keepdims=True)

    onehot = jnp.concatenate([sel1, sel2], axis=0)
    rr = lax.broadcasted_iota(jnp.int32, (RCH, RCH), 0)
    rc = lax.broadcasted_iota(jnp.int32, (RCH, RCH), 1)
    tril = (rc < rr).astype(jnp.float32)
    base = jnp.zeros((1, E), jnp.float32)
    for c in range(NRCH):
        oc = onehot[c * RCH:(c + 1) * RCH, :]
        run = lax.dot_general(tril, oc, (((1,), (0,)), ((), ())),
                              preferred_element_type=jnp.float32) + base
        base = base + jnp.sum(oc, axis=0, keepdims=True)
        rank = jnp.sum(run * oc, axis=-1, keepdims=True)
        st = jnp.sum(start * oc, axis=-1, keepdims=True)
        pos_ref[c * RCH:(c + 1) * RCH, :] = (rank + st).astype(jnp.int32)

    ie8 = lax.broadcasted_iota(jnp.int32, (1, E), 1).astype(jnp.float32)
    present = (pc > 0).astype(jnp.float32)
    lastp = jnp.max(jnp.where(pc > 0, ie8, -1.0), axis=-1,
                    keepdims=True)

    ti = (lax.broadcasted_iota(jnp.int32, (1, 128), 1) * TM).astype(
        jnp.float32)
    te = jnp.zeros((1, 128), jnp.float32)
    for e in range(E):
        te = te + (ti >= pend[:, e:e + 1]).astype(jnp.float32)
    te = jnp.minimum(te, float(E - 1))
    te = jnp.where(ti < pend_total, te, lastp)

    startmap = jnp.zeros((1, 128), jnp.float32)
    rankmap = jnp.zeros((1, 128), jnp.float32)
    nexte = jnp.full((1, 128), float(E), jnp.float32)
    for e in range(E):
        sel = (te == float(e)).astype(jnp.float32)
        startmap = startmap + sel * start[:, e:e + 1]
        rankmap = rankmap + jnp.where(
            (present[:, e:e + 1] > 0) & (te >= float(e)), 1.0, 0.0)
        cand = jnp.where((present[:, e:e + 1] > 0) & (te < float(e)),
                         float(e), float(E))
        nexte = jnp.minimum(nexte, cand)
    nexte = jnp.where(nexte == float(E), te, nexte)
    slot = rankmap - 1.0
    slot = slot - 2.0 * jnp.floor(slot * 0.5)
    runstart = (ti == startmap).astype(jnp.float32)
    prestart = runstart * (te != lastp).astype(jnp.float32)

    meta_ref[0:1, :] = te.astype(jnp.int32)
    meta_ref[1:2, :] = slot.astype(jnp.int32)
    meta_ref[2:3, :] = nexte.astype(jnp.int32)
    meta_ref[3:4, :] = runstart.astype(jnp.int32)
    meta_ref[4:5, :] = prestart.astype(jnp.int32)


def _dispatch_body(x_hbm, pos_hbm, gcat_hbm, xg_hbm, gq_hbm,
                   idx_v, xbuf, gbuf, sem):
    wid = lax.axis_index("s") * NC + lax.axis_index("c")
    for sub in range(APW // DCH):
        j0 = wid * APW + sub * DCH
        t0 = lax.rem(j0, T)
        pltpu.sync_copy(pos_hbm.at[pl.ds(j0, DCH)], idx_v)
        pltpu.sync_copy(x_hbm.at[pl.ds(t0, DCH)], xbuf)
        pltpu.sync_copy(gcat_hbm.at[pl.ds(j0, DCH)], gbuf)
        pltpu.async_copy(xbuf, xg_hbm.at[idx_v], sem).wait()
        pltpu.async_copy(gbuf, gq_hbm.at[idx_v], sem).wait()


def _combine_body(og_hbm, p1_hbm, p2_hbm, y_hbm, i1v, i2v, b1, b2, s1, s2):
    wid = lax.axis_index("s") * NC + lax.axis_index("c")
    for sub in range(TPW // CCH):
        t0 = wid * TPW + sub * CCH
        pltpu.sync_copy(p1_hbm.at[pl.ds(t0, CCH)], i1v)
        pltpu.sync_copy(p2_hbm.at[pl.ds(t0, CCH)], i2v)
        c1 = pltpu.async_copy(og_hbm.at[i1v], b1, s1)
        c2 = pltpu.async_copy(og_hbm.at[i2v], b2, s2)
        c1.wait()
        c2.wait()

        def row_body(r, carry):
            def col_body(cc, carry2):
                off = cc * 64
                for u in range(4):
                    sl = pl.ds(off + u * 16, 16)
                    b1[r, sl] = b1[r, sl] + b2[r, sl]
                return carry2

            return lax.fori_loop(0, D // 64, col_body, carry)

        lax.fori_loop(0, CCH, row_body, 0)
        pltpu.sync_copy(b1, y_hbm.at[pl.ds(t0, CCH)])


@functools.lru_cache(maxsize=None)
def _sc_kernels():
    mesh = plsc.VectorSubcoreMesh(core_axis_name="c", subcore_axis_name="s")
    dispatch = pl.kernel(
        _dispatch_body,
        out_type=(
            jax.ShapeDtypeStruct((P, D), jnp.float32),
            jax.ShapeDtypeStruct((P, 128), jnp.float32),
        ),
        mesh=mesh,
        scratch_types=[
            pltpu.VMEM((DCH,), jnp.int32),
            pltpu.VMEM((DCH, D), jnp.float32),
            pltpu.VMEM((DCH, 128), jnp.float32),
            pltpu.SemaphoreType.DMA,
        ],
    )
    combine = pl.kernel(
        _combine_body,
        out_type=jax.ShapeDtypeStruct((T, D), jnp.float32),
        mesh=mesh,
        scratch_types=[
            pltpu.VMEM((CCH,), jnp.int32),
            pltpu.VMEM((CCH,), jnp.int32),
            pltpu.VMEM((CCH, D), jnp.float32),
            pltpu.VMEM((CCH, D), jnp.float32),
            pltpu.SemaphoreType.DMA,
            pltpu.SemaphoreType.DMA,
        ],
    )
    return dispatch, combine


def _gmm_body(meta_ref, xg_ref, w1_hbm, w2_hbm, b1_ref, b2_ref, gq_ref,
              og_ref, w1a, w1b, w2a, w2b, sw1a, sw1b, sw2a, sw2b):
    i = pl.program_id(0)
    e = meta_ref[0, i]
    slot = meta_ref[1, i]
    nxt = meta_ref[2, i]
    rs = meta_ref[3, i]
    ps = meta_ref[4, i]

    @pl.when(i == 0)
    def _():
        pltpu.make_async_copy(w1_hbm.at[e], w1a, sw1a).start()
        pltpu.make_async_copy(w2_hbm.at[e], w2a, sw2a).start()

    @pl.when((ps == 1) & (slot == 0))
    def _():
        pltpu.make_async_copy(w1_hbm.at[nxt], w1b, sw1b).start()
        pltpu.make_async_copy(w2_hbm.at[nxt], w2b, sw2b).start()

    @pl.when((ps == 1) & (slot == 1))
    def _():
        pltpu.make_async_copy(w1_hbm.at[nxt], w1a, sw1a).start()
        pltpu.make_async_copy(w2_hbm.at[nxt], w2a, sw2a).start()

    @pl.when((rs == 1) & (slot == 0))
    def _():
        pltpu.make_async_copy(w1_hbm.at[e], w1a, sw1a).wait()
        pltpu.make_async_copy(w2_hbm.at[e], w2a, sw2a).wait()

    @pl.when((rs == 1) & (slot == 1))
    def _():
        pltpu.make_async_copy(w1_hbm.at[e], w1b, sw1b).wait()
        pltpu.make_async_copy(w2_hbm.at[e], w2b, sw2b).wait()

    def compute(w1buf, w2buf):
        xb = xg_ref[...].astype(jnp.bfloat16)
        h = lax.dot_general(xb, w1buf[...].astype(jnp.bfloat16),
                            (((1,), (1,)), ((), ())),
                            preferred_element_type=jnp.float32)
        h = jnp.maximum(h + b1_ref[0], 0.0).astype(jnp.bfloat16)
        o = lax.dot_general(h, w2buf[...].astype(jnp.bfloat16),
                            (((1,), (1,)), ((), ())),
                            preferred_element_type=jnp.float32)
        og_ref[...] = (o + b2_ref[0]) * gq_ref[:, 0:1]

    @pl.when(slot == 0)
    def _():
        compute(w1a, w2a)

    @pl.when(slot == 1)
    def _():
        compute(w1b, w2b)


@jax.jit
def _moe(x, Wr, W1, b1, W2, b2):
    pos, gcat, meta, laux = pl.pallas_call(
        _router_body,
        out_shape=(
            jax.ShapeDtypeStruct((K * T, 1), jnp.int32),
            jax.ShapeDtypeStruct((K * T, 128), jnp.float32),
            jax.ShapeDtypeStruct((5, 128), jnp.int32),
            jax.ShapeDtypeStruct((1, 1), jnp.float32),
        ),
    )(x, Wr)

    dispatch, combine = _sc_kernels()
    pos_flat = pos.reshape(K * T)
    xg, gq = dispatch(x, pos_flat, gcat)

    grid_spec = pltpu.PrefetchScalarGridSpec(
        num_scalar_prefetch=1,
        grid=(NT,),
        in_specs=[
            pl.BlockSpec((TM, D), lambda i, m: (i, 0)),
            pl.BlockSpec(memory_space=pl.MemorySpace.ANY),
            pl.BlockSpec(memory_space=pl.MemorySpace.ANY),
            pl.BlockSpec((1, 1, F), lambda i, m: (m[0, i], 0, 0)),
            pl.BlockSpec((1, 1, D), lambda i, m: (m[0, i], 0, 0)),
            pl.BlockSpec((TM, 128), lambda i, m: (i, 0)),
        ],
        out_specs=pl.BlockSpec((TM, D), lambda i, m: (i, 0)),
        scratch_shapes=[
            pltpu.VMEM((F, D), jnp.float32),
            pltpu.VMEM((F, D), jnp.float32),
            pltpu.VMEM((D, F), jnp.float32),
            pltpu.VMEM((D, F), jnp.float32),
            pltpu.SemaphoreType.DMA,
            pltpu.SemaphoreType.DMA,
            pltpu.SemaphoreType.DMA,
            pltpu.SemaphoreType.DMA,
        ],
    )
    og = pl.pallas_call(
        _gmm_body,
        grid_spec=grid_spec,
        out_shape=jax.ShapeDtypeStruct((P, D), jnp.float32),
    )(meta, xg, W1, W2, b1.reshape(E, 1, F), b2.reshape(E, 1, D), gq)

    y = combine(og, pos_flat[:T], pos_flat[T:])
    return y, laux[0, 0]


def kernel(x, Wr, W1, b1, W2, b2):
    return _moe(x, Wr, W1, b1, W2, b2)


# speedup vs baseline: 2.4729x; 1.0267x over previous
import functools

import jax
import jax.numpy as jnp
from jax import lax
from jax.experimental import pallas as pl
from jax.experimental.pallas import tpu as pltpu
from jax.experimental.pallas import tpu_sc as plsc

T, D, F, E, K = 2048, 1024, 2048, 8, 2
TM = 256
P = K * T + E * TM
NT = P // TM
RCH = 512
NRCH = (K * T) // RCH

NC, NS = 2, 16
NW = NC * NS
APW = (K * T) // NW
DCH = 32
TPW = T // NW
CCH = 16


def _router_body(x_ref, wr_ref, pos_ref, gcat_ref, meta_ref, laux_ref):
    x = x_ref[...]
    wr = wr_ref[...]
    logits = lax.dot_general(x, wr, (((1,), (1,)), ((), ())),
                             preferred_element_type=jnp.float32)
    m = jnp.max(logits, axis=-1, keepdims=True)
    ex = jnp.exp(logits - m)
    probs = ex / jnp.sum(ex, axis=-1, keepdims=True)

    iota = lax.broadcasted_iota(jnp.int32, (T, E), 1)
    m1 = jnp.max(probs, axis=-1, keepdims=True)
    i1 = jnp.min(jnp.where(probs == m1, iota, E), axis=-1, keepdims=True)
    masked = jnp.where(iota == i1, -1.0, probs)
    m2 = jnp.max(masked, axis=-1, keepdims=True)
    i2 = jnp.min(jnp.where(masked == m2, iota, E), axis=-1, keepdims=True)
    denom = m1 + m2
    gcat_ref[0:T, :] = jnp.broadcast_to(m1 / denom, (T, 128))
    gcat_ref[T:K * T, :] = jnp.broadcast_to(m2 / denom, (T, 128))

    sel1 = (iota == i1).astype(jnp.float32)
    sel2 = (iota == i2).astype(jnp.float32)

    f = jnp.sum(sel1 + sel2, axis=0, keepdims=True) / T
    p = jnp.sum(probs, axis=0, keepdims=True) / T
    laux_ref[...] = jnp.sum(E * f * p, axis=-1, keepdims=True)

    counts = jnp.sum(sel1, axis=0, keepdims=True) + jnp.sum(
        sel2, axis=0, keepdims=True)
    pc = jnp.floor((counts + (TM - 1)) / TM) * TM
    er = lax.broadcasted_iota(jnp.int32, (E, E), 0)
    ec = lax.broadcasted_iota(jnp.int32, (E, E), 1)
    upper = (er < ec).astype(jnp.float32)
    start = lax.dot_general(pc, upper, (((1,), (0,)), ((), ())),
                            preferred_element_type=jnp.float32)
    pend = start + pc
    pend_total = jnp.sum(pc, axis=-1, keepdims=True)

    onehot = jnp.concatenate([sel1, sel2], axis=0)
    rr = lax.broadcasted_iota(jnp.int32, (RCH, RCH), 0)
    rc = lax.broadcasted_iota(jnp.int32, (RCH, RCH), 1)
    tril = (rc < rr).astype(jnp.float32)
    base = jnp.zeros((1, E), jnp.float32)
    for c in range(NRCH):
        oc = onehot[c * RCH:(c + 1) * RCH, :]
        run = lax.dot_general(tril, oc, (((1,), (0,)), ((), ())),
                              preferred_element_type=jnp.float32) + base
        base = base + jnp.sum(oc, axis=0, keepdims=True)
        rank = jnp.sum(run * oc, axis=-1, keepdims=True)
        st = jnp.sum(start * oc, axis=-1, keepdims=True)
        pos_ref[c * RCH:(c + 1) * RCH, :] = (rank + st).astype(jnp.int32)

    ie8 = lax.broadcasted_iota(jnp.int32, (1, E), 1).astype(jnp.float32)
    present = (pc > 0).astype(jnp.float32)
    lastp = jnp.max(jnp.where(pc > 0, ie8, -1.0), axis=-1,
                    keepdims=True)

    ti = (lax.broadcasted_iota(jnp.int32, (1, 128), 1) * TM).astype(
        jnp.float32)
    te = jnp.zeros((1, 128), jnp.float32)
    for e in range(E):
        te = te + (ti >= pend[:, e:e + 1]).astype(jnp.float32)
    te = jnp.minimum(te, float(E - 1))
    te = jnp.where(ti < pend_total, te, lastp)

    startmap = jnp.zeros((1, 128), jnp.float32)
    rankmap = jnp.zeros((1, 128), jnp.float32)
    nexte = jnp.full((1, 128), float(E), jnp.float32)
    for e in range(E):
        sel = (te == float(e)).astype(jnp.float32)
        startmap = startmap + sel * start[:, e:e + 1]
        rankmap = rankmap + jnp.where(
            (present[:, e:e + 1] > 0) & (te >= float(e)), 1.0, 0.0)
        cand = jnp.where((present[:, e:e + 1] > 0) & (te < float(e)),
                         float(e), float(E))
        nexte = jnp.minimum(nexte, cand)
    nexte = jnp.where(nexte == float(E), te, nexte)
    slot = rankmap - 1.0
    slot = slot - 2.0 * jnp.floor(slot * 0.5)
    runstart = (ti == startmap).astype(jnp.float32)
    prestart = runstart * (te != lastp).astype(jnp.float32)

    meta_ref[0:1, :] = te.astype(jnp.int32)
    meta_ref[1:2, :] = slot.astype(jnp.int32)
    meta_ref[2:3, :] = nexte.astype(jnp.int32)
    meta_ref[3:4, :] = runstart.astype(jnp.int32)
    meta_ref[4:5, :] = prestart.astype(jnp.int32)


def _dispatch_body(x_hbm, pos_hbm, gcat_hbm, xg_hbm, gq_hbm,
                   idx0, idx1, xb0, xb1, gbuf, sin0, sin1, ssc):
    wid = lax.axis_index("s") * NC + lax.axis_index("c")
    nsub = APW // DCH
    idxs, xbs, sins = [idx0, idx1], [xb0, xb1], [sin0, sin1]

    def jof(s):
        return wid * APW + s * DCH

    ins = {}
    pltpu.sync_copy(pos_hbm.at[pl.ds(jof(0), DCH)], idx0)
    ins[0] = pltpu.async_copy(
        x_hbm.at[pl.ds(lax.rem(jof(0), T), DCH)], xb0, sin0)
    for s in range(nsub):
        b, nb = s % 2, (s + 1) % 2
        if s + 1 < nsub:
            pltpu.sync_copy(pos_hbm.at[pl.ds(jof(s + 1), DCH)], idxs[nb])
            ins[s + 1] = pltpu.async_copy(
                x_hbm.at[pl.ds(lax.rem(jof(s + 1), T), DCH)], xbs[nb],
                sins[nb])
        ins[s].wait()
        pltpu.sync_copy(gcat_hbm.at[pl.ds(jof(s), DCH)], gbuf)
        pltpu.async_copy(xbs[b], xg_hbm.at[idxs[b]], ssc).wait()
        pltpu.async_copy(gbuf, gq_hbm.at[idxs[b]], ssc).wait()


def _combine_body(og_hbm, p1_hbm, p2_hbm, y_hbm,
                  i1a, i1b, i2a, i2b, b1a, b1b, b2a, b2b,
                  s1a, s1b, s2a, s2b):
    wid = lax.axis_index("s") * NC + lax.axis_index("c")
    nsub = TPW // CCH
    i1s, i2s = [i1a, i1b], [i2a, i2b]
    b1s, b2s = [b1a, b1b], [b2a, b2b]
    s1s, s2s = [s1a, s1b], [s2a, s2b]

    def tof(s):
        return wid * TPW + s * CCH

    g1, g2 = {}, {}
    pltpu.sync_copy(p1_hbm.at[pl.ds(tof(0), CCH)], i1a)
    pltpu.sync_copy(p2_hbm.at[pl.ds(tof(0), CCH)], i2a)
    g1[0] = pltpu.async_copy(og_hbm.at[i1a], b1a, s1a)
    g2[0] = pltpu.async_copy(og_hbm.at[i2a], b2a, s2a)
    for s in range(nsub):
        b, nb = s % 2, (s + 1) % 2
        if s + 1 < nsub:
            pltpu.sync_copy(p1_hbm.at[pl.ds(tof(s + 1), CCH)], i1s[nb])
            pltpu.sync_copy(p2_hbm.at[pl.ds(tof(s + 1), CCH)], i2s[nb])
            g1[s + 1] = pltpu.async_copy(og_hbm.at[i1s[nb]], b1s[nb],
                                         s1s[nb])
            g2[s + 1] = pltpu.async_copy(og_hbm.at[i2s[nb]], b2s[nb],
                                         s2s[nb])
        g1[s].wait()
        g2[s].wait()
        b1, b2 = b1s[b], b2s[b]

        def row_body(r, carry):
            def col_body(cc, carry2):
                off = cc * 64
                for u in range(4):
                    sl = pl.ds(off + u * 16, 16)
                    b1[r, sl] = b1[r, sl] + b2[r, sl]
                return carry2

            return lax.fori_loop(0, D // 64, col_body, carry)

        lax.fori_loop(0, CCH, row_body, 0)
        pltpu.sync_copy(b1, y_hbm.at[pl.ds(tof(s), CCH)])


@functools.lru_cache(maxsize=None)
def _sc_kernels():
    mesh = plsc.VectorSubcoreMesh(core_axis_name="c", subcore_axis_name="s")
    dispatch = pl.kernel(
        _dispatch_body,
        out_type=(
            jax.ShapeDtypeStruct((P, D), jnp.float32),
            jax.ShapeDtypeStruct((P, 128), jnp.float32),
        ),
        mesh=mesh,
        scratch_types=[
            pltpu.VMEM((DCH,), jnp.int32),
            pltpu.VMEM((DCH,), jnp.int32),
            pltpu.VMEM((DCH, D), jnp.float32),
            pltpu.VMEM((DCH, D), jnp.float32),
            pltpu.VMEM((DCH, 128), jnp.float32),
            pltpu.SemaphoreType.DMA,
            pltpu.SemaphoreType.DMA,
            pltpu.SemaphoreType.DMA,
        ],
    )
    combine = pl.kernel(
        _combine_body,
        out_type=jax.ShapeDtypeStruct((T, D), jnp.float32),
        mesh=mesh,
        scratch_types=[
            pltpu.VMEM((CCH,), jnp.int32),
            pltpu.VMEM((CCH,), jnp.int32),
            pltpu.VMEM((CCH,), jnp.int32),
            pltpu.VMEM((CCH,), jnp.int32),
            pltpu.VMEM((CCH, D), jnp.float32),
            pltpu.VMEM((CCH, D), jnp.float32),
            pltpu.VMEM((CCH, D), jnp.float32),
            pltpu.VMEM((CCH, D), jnp.float32),
            pltpu.SemaphoreType.DMA,
            pltpu.SemaphoreType.DMA,
            pltpu.SemaphoreType.DMA,
            pltpu.SemaphoreType.DMA,
        ],
    )
    return dispatch, combine


def _gmm_body(meta_ref, xg_ref, w1_hbm, w2_hbm, b1_ref, b2_ref, gq_ref,
              og_ref, w1a, w1b, w2a, w2b, sw1a, sw1b, sw2a, sw2b):
    i = pl.program_id(0)
    e = meta_ref[0, i]
    slot = meta_ref[1, i]
    nxt = meta_ref[2, i]
    rs = meta_ref[3, i]
    ps = meta_ref[4, i]

    @pl.when(i == 0)
    def _():
        pltpu.make_async_copy(w1_hbm.at[e], w1a, sw1a).start()
        pltpu.make_async_copy(w2_hbm.at[e], w2a, sw2a).start()

    @pl.when((ps == 1) & (slot == 0))
    def _():
        pltpu.make_async_copy(w1_hbm.at[nxt], w1b, sw1b).start()
        pltpu.make_async_copy(w2_hbm.at[nxt], w2b, sw2b).start()

    @pl.when((ps == 1) & (slot == 1))
    def _():
        pltpu.make_async_copy(w1_hbm.at[nxt], w1a, sw1a).start()
        pltpu.make_async_copy(w2_hbm.at[nxt], w2a, sw2a).start()

    @pl.when((rs == 1) & (slot == 0))
    def _():
        pltpu.make_async_copy(w1_hbm.at[e], w1a, sw1a).wait()
        pltpu.make_async_copy(w2_hbm.at[e], w2a, sw2a).wait()

    @pl.when((rs == 1) & (slot == 1))
    def _():
        pltpu.make_async_copy(w1_hbm.at[e], w1b, sw1b).wait()
        pltpu.make_async_copy(w2_hbm.at[e], w2b, sw2b).wait()

    def compute(w1buf, w2buf):
        xb = xg_ref[...].astype(jnp.bfloat16)
        h = lax.dot_general(xb, w1buf[...].astype(jnp.bfloat16),
                            (((1,), (1,)), ((), ())),
                            preferred_element_type=jnp.float32)
        h = jnp.maximum(h + b1_ref[0], 0.0).astype(jnp.bfloat16)
        o = lax.dot_general(h, w2buf[...].astype(jnp.bfloat16),
                            (((1,), (1,)), ((), ())),
                            preferred_element_type=jnp.float32)
        og_ref[...] = (o + b2_ref[0]) * gq_ref[:, 0:1]

    @pl.when(slot == 0)
    def _():
        compute(w1a, w2a)

    @pl.when(slot == 1)
    def _():
        compute(w1b, w2b)


@jax.jit
def _moe(x, Wr, W1, b1, W2, b2):
    pos, gcat, meta, laux = pl.pallas_call(
        _router_body,
        out_shape=(
            jax.ShapeDtypeStruct((K * T, 1), jnp.int32),
            jax.ShapeDtypeStruct((K * T, 128), jnp.float32),
            jax.ShapeDtypeStruct((5, 128), jnp.int32),
            jax.ShapeDtypeStruct((1, 1), jnp.float32),
        ),
    )(x, Wr)

    dispatch, combine = _sc_kernels()
    pos_flat = pos.reshape(K * T)
    xg, gq = dispatch(x, pos_flat, gcat)

    grid_spec = pltpu.PrefetchScalarGridSpec(
        num_scalar_prefetch=1,
        grid=(NT,),
        in_specs=[
            pl.BlockSpec((TM, D), lambda i, m: (i, 0)),
            pl.BlockSpec(memory_space=pl.MemorySpace.ANY),
            pl.BlockSpec(memory_space=pl.MemorySpace.ANY),
            pl.BlockSpec((1, 1, F), lambda i, m: (m[0, i], 0, 0)),
            pl.BlockSpec((1, 1, D), lambda i, m: (m[0, i], 0, 0)),
            pl.BlockSpec((TM, 128), lambda i, m: (i, 0)),
        ],
        out_specs=pl.BlockSpec((TM, D), lambda i, m: (i, 0)),
        scratch_shapes=[
            pltpu.VMEM((F, D), jnp.float32),
            pltpu.VMEM((F, D), jnp.float32),
            pltpu.VMEM((D, F), jnp.float32),
            pltpu.VMEM((D, F), jnp.float32),
            pltpu.SemaphoreType.DMA,
            pltpu.SemaphoreType.DMA,
            pltpu.SemaphoreType.DMA,
            pltpu.SemaphoreType.DMA,
        ],
    )
    og = pl.pallas_call(
        _gmm_body,
        grid_spec=grid_spec,
        out_shape=jax.ShapeDtypeStruct((P, D), jnp.float32),
    )(meta, xg, W1, W2, b1.reshape(E, 1, F), b2.reshape(E, 1, D), gq)

    y = combine(og, pos_flat[:T], pos_flat[T:])
    return y, laux[0, 0]


def kernel(x, Wr, W1, b1, W2, b2):
    return _moe(x, Wr, W1, b1, W2, b2)
